# Initial kernel scaffold; baseline (speedup 1.0000x reference)
import jax
import jax.numpy as jnp
from jax.experimental import pallas as pl


def kernel(x, pos, batch, edge_attr, normal):
    raise NotImplementedError("write your pallas kernel here")


# speedup vs baseline: 4.6301x; 4.6301x over previous
import functools

import jax
import jax.numpy as jnp
from jax import lax
from jax.experimental import pallas as pl
from jax.experimental.pallas import tpu as pltpu
from jax.experimental.pallas import tpu_sc as plsc

N = 10000
K = 64
D = 128
NP = 10240
BA = 128
NR = 10112
NW = 32
NODES = 10016
NPW = NODES // NW
BC = 80


def _knn_block(pos_ref, post_ref, sq_ref, idxt_ref, neg_ref):
    bi = pl.program_id(0)
    pos_b = pos_ref[...]
    sq_b = jnp.sum(pos_b * pos_b, axis=1, keepdims=True)
    d2 = sq_b + sq_ref[...] - 2.0 * jnp.dot(
        pos_b, post_ref[...], preferred_element_type=jnp.float32)
    gids = bi * BA + lax.broadcasted_iota(jnp.int32, (BA, 1), 0)
    cols = lax.broadcasted_iota(jnp.int32, (BA, NP), 1)
    d2 = jnp.where((cols == gids) & (gids < N), -1.0, d2)
    neg_ref[...] = -d2

    def body(k, carry):
        neg = neg_ref[...]
        m = jnp.max(neg, axis=1, keepdims=True)
        p = jnp.min(jnp.where(neg == m, cols, NP), axis=1, keepdims=True)
        neg_ref[...] = jnp.where(cols == p, -jnp.inf, neg)
        idxt_ref[pl.ds(k, 1), :] = jnp.transpose(p.astype(jnp.float32), (1, 0))
        return carry

    lax.fori_loop(0, K, body, 0)


def _sc_gather(idx_hbm, xpn_hbm, xmax_hbm, pnj_hbm,
               idx_v, xrows, pnrows, acc, sem):
    wid = lax.axis_index("s") * 2 + lax.axis_index("c")
    base = wid * NPW

    def body(n, carry):
        node = base + n
        pltpu.sync_copy(idx_hbm.at[node], idx_v)
        pltpu.async_copy(xpn_hbm.at[idx_v], xrows, sem).wait()
        for c in range(D // 16):
            acc[pl.ds(c * 16, 16)] = xrows[0, pl.ds(c * 16, 16)]

        def rbody(r, rc):
            for c in range(D // 16):
                sl = pl.ds(c * 16, 16)
                acc[sl] = jnp.maximum(acc[sl], xrows[r, sl])
            return rc

        lax.fori_loop(1, K, rbody, 0)

        def pbody(r, rc):
            pnrows[r, :] = xrows[r, pl.ds(D, 16)]
            return rc

        lax.fori_loop(0, K, pbody, 0)
        pltpu.sync_copy(acc, xmax_hbm.at[node])
        pltpu.sync_copy(pnrows, pnj_hbm.at[node])
        return carry

    lax.fori_loop(0, NPW, body, 0)


def _angles(u0, u1, u2, v0, v1, v2):
    c0 = u1 * v2 - u2 * v1
    c1 = u2 * v0 - u0 * v2
    c2 = u0 * v1 - u1 * v0
    cn = jnp.sqrt(c0 * c0 + c1 * c1 + c2 * c2 + 1e-12)
    dt = u0 * v0 + u1 * v1 + u2 * v2
    hyp = jnp.sqrt(cn * cn + dt * dt)
    return cn / hyp, dt / hyp


def _ppf_block(xmax_ref, pn_ref, pnj_ref, ea_ref, eas_ref, out_ref):
    pni = pn_ref[...]
    pj = pnj_ref[...]
    ps = [pj[:, :, c] - pni[:, c:c + 1] for c in range(3)]
    nj = [pj[:, :, 3 + c] for c in range(3)]
    ni = [pni[:, 3 + c:4 + c] for c in range(3)]
    s1, c1 = _angles(ni[0], ni[1], ni[2], ps[0], ps[1], ps[2])
    s2, c2 = _angles(nj[0], nj[1], nj[2], ps[0], ps[1], ps[2])
    s3, c3 = _angles(ni[0], ni[1], ni[2], nj[0], nj[1], nj[2])
    dist = jnp.sqrt(ps[0] * ps[0] + ps[1] * ps[1] + ps[2] * ps[2] + 1e-12)
    feats = [dist * 0.5, s1, c1, s2, c2, s3, c3]
    fmax = jnp.concatenate(
        [jnp.max(f, axis=1, keepdims=True) for f in feats], axis=1)
    ea = eas_ref[...]
    for r in range(K - 1):
        ea = jnp.maximum(ea, ea_ref[:, r, :])
    out_ref[:, 0:D] = xmax_ref[...]
    out_ref[:, D:D + 7] = fmax
    out_ref[:, D + 7:D + 11] = ea


def _stage_a(pos):
    pos4 = jnp.pad(pos, ((0, NR - N), (0, 1)))
    post = jnp.pad(pos.T, ((0, 1), (0, NP - N)))
    sq = jnp.pad(jnp.sum(pos * pos, axis=1)[None, :], ((0, 0), (0, NP - N)),
                 constant_values=1e30)
    idxt = pl.pallas_call(
        _knn_block,
        grid=(NR // BA,),
        in_specs=[
            pl.BlockSpec((BA, 4), lambda i: (i, 0)),
            pl.BlockSpec((4, NP), lambda i: (0, 0)),
            pl.BlockSpec((1, NP), lambda i: (0, 0)),
        ],
        out_specs=pl.BlockSpec((K, BA), lambda i: (0, i)),
        out_shape=jax.ShapeDtypeStruct((K, NR), jnp.float32),
        scratch_shapes=[pltpu.VMEM((BA, NP), jnp.float32)],
    )(pos4, post, sq)
    return idxt.T[:NODES].astype(jnp.int32)


def _stage_b(idx, xpn):
    mesh = plsc.VectorSubcoreMesh(core_axis_name="c", subcore_axis_name="s")
    f = pl.kernel(
        _sc_gather, mesh=mesh,
        out_type=[
            jax.ShapeDtypeStruct((NODES, D), jnp.float32),
            jax.ShapeDtypeStruct((NODES, K, 16), jnp.float32),
        ],
        scratch_types=[
            pltpu.VMEM((K,), jnp.int32),
            pltpu.VMEM((K, 2 * D), jnp.float32),
            pltpu.VMEM((K, 16), jnp.float32),
            pltpu.VMEM((D,), jnp.float32),
            pltpu.SemaphoreType.DMA,
        ],
    )
    return f(idx, xpn)


def _stage_c(xmax, pn, pnj, ea3, eas):
    return pl.pallas_call(
        _ppf_block,
        grid=(N // BC,),
        in_specs=[
            pl.BlockSpec((BC, D), lambda i: (i, 0)),
            pl.BlockSpec((BC, 8), lambda i: (i, 0)),
            pl.BlockSpec((BC, K, 16), lambda i: (i, 0, 0)),
            pl.BlockSpec((BC, K - 1, 4), lambda i: (i, 0, 0)),
            pl.BlockSpec((BC, 4), lambda i: (i, 0)),
        ],
        out_specs=pl.BlockSpec((BC, D + 11), lambda i: (i, 0)),
        out_shape=jax.ShapeDtypeStruct((N, D + 11), jnp.float32),
    )(xmax, pn, pnj, ea3, eas)


def kernel(x, pos, batch, edge_attr, normal):
    idx = _stage_a(pos)
    pn = jnp.pad(jnp.concatenate([pos, normal], axis=1), ((0, 0), (0, 2)))
    xpn = jnp.pad(jnp.concatenate([x, pos, normal], axis=1),
                  ((0, 0), (0, 2 * D - (D + 6))))
    xmax, pnj = _stage_b(idx, xpn)
    ea3 = edge_attr[:N * (K - 1)].reshape(N, K - 1, 4)
    eas = edge_attr[N * (K - 1):]
    out = _stage_c(xmax[:N], pn, pnj[:N], ea3, eas)
    return (out, pos, batch)
